# baseline (speedup 1.0000x reference)
import functools

import jax
import jax.numpy as jnp
from jax.experimental import pallas as pl

G, T, H, E, C = 2, 2048, 2048, 16, 128
TBLK_A = 512
TBLK_C = 256
NEG_INF = float("-inf")


def _probs_body(x_ref, w_ref, b_ref, probs_ref, z_ref):
    g = pl.program_id(0)
    tb = pl.program_id(1)
    x = x_ref[0]
    w = w_ref[...]
    b = b_ref[...]
    logits = jax.lax.dot_general(
        w, x, dimension_numbers=(((0,), (1,)), ((), ())),
        preferred_element_type=jnp.float32)
    logits = logits + b.reshape(E, 1)
    m = jnp.max(logits, axis=0, keepdims=True)
    ex = jnp.exp(logits - m)
    s = jnp.sum(ex, axis=0, keepdims=True)
    probs_ref[0] = ex / s
    lse = m + jnp.log(s)
    zpart = jnp.sum(lse * lse).reshape(1, 1)

    @pl.when(jnp.logical_and(g == 0, tb == 0))
    def _():
        z_ref[...] = jnp.zeros_like(z_ref)

    z_ref[...] += zpart


def _topk_body(p_ref, ei_ref, eg_ref):
    x = p_ref[...]
    rows = G * E
    iota_t = jax.lax.broadcasted_iota(jnp.int32, (rows, T), 1)
    iota_c = jax.lax.broadcasted_iota(jnp.int32, (rows, C), 1)

    def step(c, carry):
        x, ei, eg = carry
        m = jnp.max(x, axis=1, keepdims=True)
        hit = x == m
        idx = jnp.min(jnp.where(hit, iota_t, T), axis=1, keepdims=True)
        x = jnp.where(iota_t == idx, NEG_INF, x)
        colmask = iota_c == c
        ei = jnp.where(colmask, idx, ei)
        eg = jnp.where(colmask, m, eg)
        return x, ei, eg

    ei0 = jnp.zeros((rows, C), jnp.int32)
    eg0 = jnp.zeros((rows, C), jnp.float32)
    _, ei, eg = jax.lax.fori_loop(0, C, step, (x, ei0, eg0))
    ei_ref[...] = ei
    eg_ref[...] = eg


def _mask_body(ei_ref, eg_ref, disp_ref, comb_ref):
    tb = pl.program_id(1)
    t0 = tb * TBLK_C
    ti = jax.lax.broadcasted_iota(jnp.int32, (TBLK_C, E, C), 0) + t0
    hit = ei_ref[0][None, :, :] == ti
    disp_ref[0] = jnp.where(hit, 1.0, 0.0).astype(jnp.float32)
    comb_ref[0] = jnp.where(hit, eg_ref[0][None, :, :], 0.0).astype(jnp.float32)


@functools.partial(jax.jit, static_argnums=())
def _run(x, w, b):
    probs_t, zsum = pl.pallas_call(
        _probs_body,
        grid=(G, T // TBLK_A),
        in_specs=[
            pl.BlockSpec((1, TBLK_A, H), lambda g, tb: (g, tb, 0)),
            pl.BlockSpec((H, E), lambda g, tb: (0, 0)),
            pl.BlockSpec((1, E), lambda g, tb: (0, 0)),
        ],
        out_specs=[
            pl.BlockSpec((1, E, TBLK_A), lambda g, tb: (g, 0, tb)),
            pl.BlockSpec((1, 1), lambda g, tb: (0, 0)),
        ],
        out_shape=[
            jax.ShapeDtypeStruct((G, E, T), jnp.float32),
            jax.ShapeDtypeStruct((1, 1), jnp.float32),
        ],
    )(x, w, b.reshape(1, E))

    ei, eg = pl.pallas_call(
        _topk_body,
        in_specs=[pl.BlockSpec((G * E, T), lambda: (0, 0))],
        out_specs=[
            pl.BlockSpec((G * E, C), lambda: (0, 0)),
            pl.BlockSpec((G * E, C), lambda: (0, 0)),
        ],
        out_shape=[
            jax.ShapeDtypeStruct((G * E, C), jnp.int32),
            jax.ShapeDtypeStruct((G * E, C), jnp.float32),
        ],
    )(probs_t.reshape(G * E, T))

    disp, comb = pl.pallas_call(
        _mask_body,
        grid=(G, T // TBLK_C),
        in_specs=[
            pl.BlockSpec((1, E, C), lambda g, tb: (g, 0, 0)),
            pl.BlockSpec((1, E, C), lambda g, tb: (g, 0, 0)),
        ],
        out_specs=[
            pl.BlockSpec((1, TBLK_C, E, C), lambda g, tb: (g, tb, 0, 0)),
            pl.BlockSpec((1, TBLK_C, E, C), lambda g, tb: (g, tb, 0, 0)),
        ],
        out_shape=[
            jax.ShapeDtypeStruct((G, T, E, C), jnp.float32),
            jax.ShapeDtypeStruct((G, T, E, C), jnp.float32),
        ],
    )(ei.reshape(G, E, C), eg.reshape(G, E, C))

    z_loss = zsum[0, 0] / (G * T)
    return disp, comb, z_loss


def kernel(inputs, kernel, bias, expert_capacity):
    del expert_capacity
    return _run(inputs, kernel, bias)


# speedup vs baseline: 1.3762x; 1.3762x over previous
import functools

import jax
import jax.numpy as jnp
from jax.experimental import pallas as pl

G, T, H, E, C = 2, 2048, 2048, 16, 128
TBLK_A = 512
TBLK_C = 256
NEG_INF = float("-inf")


def _probs_body(x_ref, w_ref, b_ref, probs_ref, z_ref):
    g = pl.program_id(0)
    tb = pl.program_id(1)
    x = x_ref[0]
    w = w_ref[...]
    b = b_ref[...]
    logits = jax.lax.dot_general(
        w, x, dimension_numbers=(((0,), (1,)), ((), ())),
        preferred_element_type=jnp.float32)
    logits = logits + b.reshape(E, 1)
    m = jnp.max(logits, axis=0, keepdims=True)
    ex = jnp.exp(logits - m)
    s = jnp.sum(ex, axis=0, keepdims=True)
    probs_ref[0] = ex / s
    lse = m + jnp.log(s)
    zpart = jnp.sum(lse * lse).reshape(1, 1)

    @pl.when(jnp.logical_and(g == 0, tb == 0))
    def _():
        z_ref[...] = jnp.zeros_like(z_ref)

    z_ref[...] += zpart


def _first(av, ai, bv, bi):
    return (av > bv) | ((av == bv) & (ai < bi))


def _cex(v, i, lane, j, desc):
    islow = (lane & j) == 0
    pv = jnp.where(islow, jnp.roll(v, -j, 1), jnp.roll(v, j, 1))
    pi = jnp.where(islow, jnp.roll(i, -j, 1), jnp.roll(i, j, 1))
    sf = _first(v, i, pv, pi)
    keep = sf == (islow == desc)
    return jnp.where(keep, v, pv), jnp.where(keep, i, pi)


def _topk_body(p_ref, ei_ref, eg_ref):
    rows = G * E
    v = p_ref[...]
    lane = jax.lax.broadcasted_iota(jnp.int32, (rows, T), 1)
    i = lane
    want = lane < (T // 2)
    for k in (2, 4, 8, 16, 32, 64, 128):
        j = k // 2
        while j >= 1:
            if k < 128:
                desc = want ^ ((lane & k) != 0)
            else:
                desc = want
            v, i = _cex(v, i, lane, j, desc)
            j //= 2
    w = T
    while w > C:
        h = w // 2
        f = _first(v[:, :h], i[:, :h], v[:, h:w], i[:, h:w])
        v = jnp.where(f, v[:, :h], v[:, h:w])
        i = jnp.where(f, i[:, :h], i[:, h:w])
        lane_h = lane[:, :h]
        desc_h = lane_h < max(h // 2, C)
        for j in (64, 32, 16, 8, 4, 2, 1):
            v, i = _cex(v, i, lane_h, j, desc_h)
        w = h
    ei_ref[...] = i
    eg_ref[...] = v


def _mask_body(ei_ref, eg_ref, disp_ref, comb_ref):
    tb = pl.program_id(1)
    t0 = tb * TBLK_C
    ti = jax.lax.broadcasted_iota(jnp.int32, (TBLK_C, E, C), 0) + t0
    hit = ei_ref[0][None, :, :] == ti
    disp_ref[0] = jnp.where(hit, 1.0, 0.0).astype(jnp.float32)
    comb_ref[0] = jnp.where(hit, eg_ref[0][None, :, :], 0.0).astype(jnp.float32)


@functools.partial(jax.jit, static_argnums=())
def _run(x, w, b):
    probs_t, zsum = pl.pallas_call(
        _probs_body,
        grid=(G, T // TBLK_A),
        in_specs=[
            pl.BlockSpec((1, TBLK_A, H), lambda g, tb: (g, tb, 0)),
            pl.BlockSpec((H, E), lambda g, tb: (0, 0)),
            pl.BlockSpec((1, E), lambda g, tb: (0, 0)),
        ],
        out_specs=[
            pl.BlockSpec((1, E, TBLK_A), lambda g, tb: (g, 0, tb)),
            pl.BlockSpec((1, 1), lambda g, tb: (0, 0)),
        ],
        out_shape=[
            jax.ShapeDtypeStruct((G, E, T), jnp.float32),
            jax.ShapeDtypeStruct((1, 1), jnp.float32),
        ],
    )(x, w, b.reshape(1, E))

    ei, eg = pl.pallas_call(
        _topk_body,
        in_specs=[pl.BlockSpec((G * E, T), lambda: (0, 0))],
        out_specs=[
            pl.BlockSpec((G * E, C), lambda: (0, 0)),
            pl.BlockSpec((G * E, C), lambda: (0, 0)),
        ],
        out_shape=[
            jax.ShapeDtypeStruct((G * E, C), jnp.int32),
            jax.ShapeDtypeStruct((G * E, C), jnp.float32),
        ],
    )(probs_t.reshape(G * E, T))

    disp, comb = pl.pallas_call(
        _mask_body,
        grid=(G, T // TBLK_C),
        in_specs=[
            pl.BlockSpec((1, E, C), lambda g, tb: (g, 0, 0)),
            pl.BlockSpec((1, E, C), lambda g, tb: (g, 0, 0)),
        ],
        out_specs=[
            pl.BlockSpec((1, TBLK_C, E, C), lambda g, tb: (g, tb, 0, 0)),
            pl.BlockSpec((1, TBLK_C, E, C), lambda g, tb: (g, tb, 0, 0)),
        ],
        out_shape=[
            jax.ShapeDtypeStruct((G, T, E, C), jnp.float32),
            jax.ShapeDtypeStruct((G, T, E, C), jnp.float32),
        ],
    )(ei.reshape(G, E, C), eg.reshape(G, E, C))

    z_loss = zsum[0, 0] / (G * T)
    return disp, comb, z_loss


def kernel(inputs, kernel, bias, expert_capacity):
    del expert_capacity
    return _run(inputs, kernel, bias)
